# Initial kernel scaffold; baseline (speedup 1.0000x reference)
import functools

import jax
import jax.numpy as jnp
from jax import lax
from jax.experimental import pallas as pl
from jax.experimental.pallas import tpu as pltpu
from jax.experimental.pallas import tpu_sc as plsc

N = 100000
D = 128
S = 512
L = 16
NG = D // L
NC = 2
NS = 16
NW = NC * NS
CH = 3128
BLK = 391
NBLK = 8
SROWS = S // NS


def _sc_partial_sums(x, batch):
  mesh = plsc.VectorSubcoreMesh(core_axis_name="c", subcore_axis_name="s")

  @functools.partial(
      pl.kernel,
      out_type=jax.ShapeDtypeStruct((NC, S, D), jnp.float32),
      mesh=mesh,
      scratch_types=[
          pltpu.VMEM((CH,), jnp.int32),
          pltpu.VMEM((BLK, D), jnp.float32),
          pltpu.VMEM((SROWS, D), jnp.float32),
          pltpu.VMEM((1, D), jnp.float32),
          pltpu.VMEM((1,), jnp.int32),
          pltpu.VMEM_SHARED((S, D), jnp.float32),
      ],
  )
  def k(x_hbm, b_hbm, out_hbm, ids_v, xb_v, z_v, fl_v, fi_v, acc_sh):
    cid = lax.axis_index("c")
    sid = lax.axis_index("s")
    wid = cid * NS + sid
    zero = jnp.zeros((L,), jnp.float32)

    def zrow(i, carry):
      for j in range(NG):
        z_v[i, pl.ds(j * L, L)] = zero
      return carry
    lax.fori_loop(0, SROWS, zrow, 0)
    pltpu.sync_copy(z_v, acc_sh.at[pl.ds(sid * SROWS, SROWS)])
    plsc.subcore_barrier()

    base = wid * CH
    n = jnp.minimum(CH, N - base)
    idbase = jnp.minimum(base, N - CH)
    ioff = base - idbase
    pltpu.sync_copy(b_hbm.at[pl.ds(idbase, CH)], ids_v)

    def flush(a, p):
      for j in range(NG):
        fl_v[0, pl.ds(j * L, L)] = a[j]
      fi_v[0] = p
      pltpu.sync_copy(fl_v, acc_sh.at[fi_v], add=True)
      return tuple(zero for _ in range(NG))

    def block_step(b, carry):
      acc, prev = carry
      bstart = base + b * BLK
      xbase = jnp.minimum(bstart, N - BLK)
      xoff = bstart - xbase
      bn = jnp.clip(n - b * BLK, 0, BLK)
      pltpu.sync_copy(x_hbm.at[pl.ds(xbase, BLK)], xb_v)

      def row_step(r, carry2):
        acc2, prev2 = carry2
        seg = ids_v[ioff + b * BLK + r]
        acc2 = lax.cond(seg != prev2, flush, lambda a, p: a, acc2, prev2)
        row = xoff + r
        acc2 = tuple(
            acc2[j] + xb_v[row, pl.ds(j * L, L)] for j in range(NG))
        return (acc2, seg)

      return lax.fori_loop(0, bn, row_step, (acc, prev))

    acc0 = tuple(zero for _ in range(NG))
    prev0 = ids_v[ioff]
    acc, prev = lax.fori_loop(0, NBLK, block_step, (acc0, prev0))
    flush(acc, prev)

    plsc.subcore_barrier()
    pltpu.sync_copy(acc_sh.at[pl.ds(sid * SROWS, SROWS)], z_v)
    pltpu.sync_copy(z_v, out_hbm.at[cid, pl.ds(sid * SROWS, SROWS)])

  return k(x, batch)


def _combine(parts):
  def body(p_ref, o_ref):
    o_ref[...] = p_ref[0] + p_ref[1]

  return pl.pallas_call(
      body,
      out_shape=jax.ShapeDtypeStruct((S, D), jnp.float32),
  )(parts)


@jax.jit
def kernel(x, batch):
  parts = _sc_partial_sums(x, batch.astype(jnp.int32))
  return _combine(parts)


# speedup vs baseline: 2.8520x; 2.8520x over previous
import functools

import jax
import jax.numpy as jnp
from jax import lax
from jax.experimental import pallas as pl
from jax.experimental.pallas import tpu as pltpu
from jax.experimental.pallas import tpu_sc as plsc

N = 100000
D = 128
S = 512
L = 16
NG = D // L
NC = 2
NS = 16
NW = NC * NS
CH = 3128
BLK = 392
NBLK = 8
SROWS = S // NS


def _sc_partial_sums(x, batch):
  mesh = plsc.VectorSubcoreMesh(core_axis_name="c", subcore_axis_name="s")

  @functools.partial(
      pl.kernel,
      out_type=jax.ShapeDtypeStruct((NC, S, D), jnp.float32),
      mesh=mesh,
      scratch_types=[
          pltpu.VMEM((CH + L,), jnp.int32),
          pltpu.VMEM((BLK * D,), jnp.float32),
          pltpu.VMEM((SROWS, D), jnp.float32),
          pltpu.VMEM((L, D), jnp.float32),
          pltpu.VMEM((L,), jnp.int32),
          pltpu.VMEM_SHARED((S + 1, D), jnp.float32),
      ],
  )
  def k(x_hbm, b_hbm, out_hbm, ids_v, xb_v, z_v, fl_v, fi_v, acc_sh):
    cid = lax.axis_index("c")
    sid = lax.axis_index("s")
    wid = cid * NS + sid
    zero = jnp.zeros((L,), jnp.float32)

    def zrow(i, carry):
      for j in range(NG):
        z_v.at[i][pl.ds(j * L, L)] = zero
      return carry
    lax.fori_loop(0, SROWS, zrow, 0)
    pltpu.sync_copy(z_v, acc_sh.at[pl.ds(sid * SROWS, SROWS)])
    plsc.subcore_barrier()

    base = wid * CH
    n = jnp.minimum(CH, N - base)
    idbase = pl.multiple_of(jnp.minimum(base, N - CH), 8)
    ioff = base - idbase
    pltpu.sync_copy(b_hbm.at[pl.ds(idbase, CH)], ids_v.at[pl.ds(0, CH)])

    lanes = lax.iota(jnp.int32, L)
    dummy = jnp.full((L,), S, jnp.int32)
    fi_v[...] = dummy

    def scatter_batch(c):
      pltpu.sync_copy(fl_v, acc_sh.at[fi_v], add=True)
      fi_v[...] = dummy
      return jnp.int32(0)

    def flush(a, p, cnt):
      for j in range(NG):
        fl_v.at[cnt][pl.ds(j * L, L)] = a[j]
      fi_v[...] = jnp.where(lanes == cnt, p, fi_v[...])
      cnt = cnt + 1
      cnt = lax.cond(cnt == L, scatter_batch, lambda c: c, cnt)
      return tuple(zero for _ in range(NG)), cnt

    def block_step(b, carry):
      bstart = base + b * BLK
      xbase = pl.multiple_of(jnp.minimum(bstart, N - BLK), 8)
      xoff = bstart - xbase
      bn = jnp.clip(n - b * BLK, 0, BLK)
      pltpu.sync_copy(x_hbm.at[pl.ds(xbase * D, BLK * D)], xb_v)

      def row_step(r, carry2):
        acc2, prev2, cnt2 = carry2
        seg = ids_v[pl.ds(ioff + b * BLK + r, L)][0]
        acc2, cnt2 = lax.cond(
            seg != prev2, flush, lambda a, p, c: (a, c), acc2, prev2, cnt2)
        row = (xoff + r) * D
        acc2 = tuple(
            acc2[j] + xb_v[pl.ds(row + j * L, L)] for j in range(NG))
        return (acc2, seg, cnt2)

      return lax.fori_loop(0, bn, row_step, carry)

    acc0 = tuple(zero for _ in range(NG))
    prev0 = ids_v[pl.ds(ioff, L)][0]
    cnt0 = jnp.int32(0)
    acc, prev, cnt = lax.fori_loop(0, NBLK, block_step, (acc0, prev0, cnt0))
    _, cnt = flush(acc, prev, cnt)
    scatter_batch(cnt)

    plsc.subcore_barrier()
    pltpu.sync_copy(acc_sh.at[pl.ds(sid * SROWS, SROWS)], z_v)
    pltpu.sync_copy(z_v, out_hbm.at[cid, pl.ds(sid * SROWS, SROWS)])

  return k(x, batch)


def _combine(parts):
  def body(p_ref, o_ref):
    o_ref[...] = p_ref[0] + p_ref[1]

  return pl.pallas_call(
      body,
      out_shape=jax.ShapeDtypeStruct((S, D), jnp.float32),
  )(parts)


@jax.jit
def kernel(x, batch):
  parts = _sc_partial_sums(x.reshape(N * D), batch.astype(jnp.int32))
  return _combine(parts)


# speedup vs baseline: 5.9277x; 2.0784x over previous
import functools

import jax
import jax.numpy as jnp
from jax import lax
from jax.experimental import pallas as pl
from jax.experimental.pallas import tpu as pltpu
from jax.experimental.pallas import tpu_sc as plsc

N = 100000
D = 128
S = 512
L = 16
NG = D // L
NC = 2
NS = 16
NW = NC * NS
CL = 128
K = 26
SROWS = S // NS


def _sc_partial_sums(x, batch):
  mesh = plsc.VectorSubcoreMesh(core_axis_name="c", subcore_axis_name="s")

  @functools.partial(
      pl.kernel,
      out_type=jax.ShapeDtypeStruct((NC, S, D), jnp.float32),
      mesh=mesh,
      scratch_types=[
          pltpu.VMEM((K, CL), jnp.int32),
          pltpu.VMEM((CL, D), jnp.float32),
          pltpu.VMEM((CL, D), jnp.float32),
          pltpu.VMEM((SROWS, D), jnp.float32),
          pltpu.VMEM_SHARED((S + 1, D), jnp.float32),
          pltpu.SemaphoreType.DMA,
          pltpu.SemaphoreType.DMA,
          pltpu.SemaphoreType.DMA,
      ],
  )
  def k(x_hbm, b_hbm, out_hbm, ids_v, xb0, xb1, z_v, acc_sh, sid_sem,
        sem0, sem1):
    cid = lax.axis_index("c")
    sid = lax.axis_index("s")
    wid = cid * NS + sid
    zero = jnp.zeros((L,), jnp.float32)
    lanes = lax.iota(jnp.int32, L)

    def zrow(i, carry):
      for j in range(NG):
        z_v.at[i][pl.ds(j * L, L)] = zero
      return carry
    lax.fori_loop(0, SROWS, zrow, 0)
    pltpu.sync_copy(z_v, acc_sh.at[pl.ds(sid * SROWS, SROWS)])
    plsc.subcore_barrier()

    c0 = wid * K

    def chunk_start(c):
      return pl.multiple_of(jnp.minimum(c * CL, N - CL), 8)

    for j in range(K):
      pltpu.async_copy(
          b_hbm.at[pl.ds(chunk_start(c0 + j), CL)], ids_v.at[j], sid_sem)
    for j in range(K):
      pltpu.make_async_copy(
          b_hbm.at[pl.ds(0, CL)], ids_v.at[j], sid_sem).wait()
    def mask_row(j, carry):
      c = c0 + j
      st = chunk_start(c)
      for g in range(CL // L):
        gl = st + g * L + lanes
        v = ids_v.at[j][pl.ds(g * L, L)]
        ids_v.at[j][pl.ds(g * L, L)] = jnp.where(gl >= c * CL, v, S)
      return carry
    lax.fori_loop(0, K, mask_row, 0)

    bufs = (xb0, xb1)
    sems = (sem0, sem1)

    def issue(j, half):
      pltpu.async_copy(
          x_hbm.at[pl.ds(chunk_start(c0 + j), CL)], bufs[half], sems[half])

    def wait(half):
      pltpu.make_async_copy(
          x_hbm.at[pl.ds(0, CL)], bufs[half], sems[half]).wait()

    def scatter(j, half):
      pltpu.sync_copy(bufs[half], acc_sh.at[ids_v.at[j]], add=True)

    issue(0, 0)

    def pair_step(i, carry):
      j0 = 2 * i
      wait(0)
      issue(j0 + 1, 1)
      scatter(j0, 0)
      wait(1)
      issue(j0 + 2, 0)
      scatter(j0 + 1, 1)
      return carry
    lax.fori_loop(0, K // 2, pair_step, 0)
    wait(0)

    plsc.subcore_barrier()
    pltpu.sync_copy(acc_sh.at[pl.ds(sid * SROWS, SROWS)], z_v)
    pltpu.sync_copy(z_v, out_hbm.at[cid, pl.ds(sid * SROWS, SROWS)])

  return k(x, batch)


def _combine(parts):
  def body(p_ref, o_ref):
    o_ref[...] = p_ref[0] + p_ref[1]

  return pl.pallas_call(
      body,
      out_shape=jax.ShapeDtypeStruct((S, D), jnp.float32),
  )(parts)


@jax.jit
def kernel(x, batch):
  parts = _sc_partial_sums(x, batch.astype(jnp.int32))
  return _combine(parts)


# speedup vs baseline: 6.5305x; 1.1017x over previous
import functools

import jax
import jax.numpy as jnp
from jax import lax
from jax.experimental import pallas as pl
from jax.experimental.pallas import tpu as pltpu
from jax.experimental.pallas import tpu_sc as plsc

N = 100000
D = 128
S = 512
L = 16
NG = D // L
NC = 2
NS = 16
NW = NC * NS
CL = 128
K = 25
NBUF = 4
SROWS = S // NS


def _sc_partial_sums(x, batch):
  mesh = plsc.VectorSubcoreMesh(core_axis_name="c", subcore_axis_name="s")

  @functools.partial(
      pl.kernel,
      out_type=jax.ShapeDtypeStruct((NC, S, D), jnp.float32),
      mesh=mesh,
      scratch_types=[
          pltpu.VMEM((K, CL), jnp.int32),
          [pltpu.VMEM((CL, D), jnp.float32) for _ in range(NBUF)],
          pltpu.VMEM((SROWS, D), jnp.float32),
          pltpu.VMEM_SHARED((S + 1, D), jnp.float32),
          pltpu.SemaphoreType.DMA,
          [pltpu.SemaphoreType.DMA for _ in range(NBUF)],
          [pltpu.SemaphoreType.DMA for _ in range(NBUF)],
      ],
  )
  def k(x_hbm, b_hbm, out_hbm, ids_v, bufs, z_v, acc_sh, sid_sem,
        ssems, csems):
    cid = lax.axis_index("c")
    sid = lax.axis_index("s")
    wid = cid * NS + sid
    zero = jnp.zeros((L,), jnp.float32)
    lanes = lax.iota(jnp.int32, L)

    def zrow(i, carry):
      for j in range(NG):
        z_v.at[i][pl.ds(j * L, L)] = zero
      return carry
    lax.fori_loop(0, SROWS, zrow, 0)
    pltpu.sync_copy(z_v, acc_sh.at[pl.ds(sid * SROWS, SROWS)])
    plsc.subcore_barrier()

    def chunk_start(j):
      return pl.multiple_of(jnp.minimum((wid + NW * j) * CL, N - CL), 8)

    for j in range(K):
      pltpu.async_copy(
          b_hbm.at[pl.ds(chunk_start(j), CL)], ids_v.at[j], sid_sem)
    for j in range(K):
      pltpu.make_async_copy(
          b_hbm.at[pl.ds(0, CL)], ids_v.at[j], sid_sem).wait()
    def mask_row(j, carry):
      c = wid + NW * j
      st = pl.multiple_of(jnp.minimum(c * CL, N - CL), 8)
      for g in range(CL // L):
        gl = st + g * L + lanes
        v = ids_v.at[j][pl.ds(g * L, L)]
        ids_v.at[j][pl.ds(g * L, L)] = jnp.where(gl >= c * CL, v, S)
      return carry
    lax.fori_loop(0, K, mask_row, 0)

    def issue_stage(j):
      b = j % NBUF
      pltpu.async_copy(
          x_hbm.at[pl.ds(chunk_start(j), CL)], bufs[b], ssems[b])

    def wait_stage(j):
      b = j % NBUF
      pltpu.make_async_copy(
          x_hbm.at[pl.ds(0, CL)], bufs[b], ssems[b]).wait()

    def issue_scatter(j):
      b = j % NBUF
      pltpu.async_copy(bufs[b], acc_sh.at[ids_v.at[j]], csems[b], add=True)

    def wait_scatter(j):
      b = j % NBUF
      pltpu.make_async_copy(
          bufs[b], acc_sh.at[ids_v.at[j]], csems[b]).wait()

    for j in range(min(NBUF, K)):
      issue_stage(j)
    for j in range(K):
      wait_stage(j)
      issue_scatter(j)
      if j - 2 >= 0 and j + 2 < K:
        wait_scatter(j - 2)
        issue_stage(j + 2)
    for j in range(max(K - 4, 0), K):
      wait_scatter(j)

    plsc.subcore_barrier()
    pltpu.sync_copy(acc_sh.at[pl.ds(sid * SROWS, SROWS)], z_v)
    pltpu.sync_copy(z_v, out_hbm.at[cid, pl.ds(sid * SROWS, SROWS)])

  return k(x, batch)


def _combine(parts):
  def body(p_ref, o_ref):
    o_ref[...] = p_ref[0] + p_ref[1]

  return pl.pallas_call(
      body,
      out_shape=jax.ShapeDtypeStruct((S, D), jnp.float32),
  )(parts)


@jax.jit
def kernel(x, batch):
  parts = _sc_partial_sums(x, batch.astype(jnp.int32))
  return _combine(parts)


# speedup vs baseline: 6.8245x; 1.0450x over previous
import functools

import jax
import jax.numpy as jnp
from jax import lax
from jax.experimental import pallas as pl
from jax.experimental.pallas import tpu as pltpu
from jax.experimental.pallas import tpu_sc as plsc

N = 100000
D = 128
S = 512
L = 16
NG = D // L
NC = 2
NS = 16
NW = NC * NS
CL = 128
K = 25
NBUF = 6
SROWS = S // NS


def _sc_partial_sums(x, batch):
  mesh = plsc.VectorSubcoreMesh(core_axis_name="c", subcore_axis_name="s")

  @functools.partial(
      pl.kernel,
      out_type=jax.ShapeDtypeStruct((NC, S, D), jnp.float32),
      mesh=mesh,
      scratch_types=[
          pltpu.VMEM((K, CL), jnp.int32),
          [pltpu.VMEM((CL, D), jnp.float32) for _ in range(NBUF)],
          pltpu.VMEM((SROWS, D), jnp.float32),
          pltpu.VMEM_SHARED((S + 1, D), jnp.float32),
          pltpu.SemaphoreType.DMA,
          [pltpu.SemaphoreType.DMA for _ in range(NBUF)],
          [pltpu.SemaphoreType.DMA for _ in range(NBUF)],
      ],
  )
  def k(x_hbm, b_hbm, out_hbm, ids_v, bufs, z_v, acc_sh, sid_sem,
        ssems, csems):
    cid = lax.axis_index("c")
    sid = lax.axis_index("s")
    wid = cid * NS + sid
    zero = jnp.zeros((L,), jnp.float32)
    lanes = lax.iota(jnp.int32, L)

    def chunk_start(j):
      return pl.multiple_of(jnp.minimum((wid + NW * j) * CL, N - CL), 8)

    for j in range(K):
      pltpu.async_copy(
          b_hbm.at[pl.ds(chunk_start(j), CL)], ids_v.at[j], sid_sem)

    def zrow(i, carry):
      for j in range(NG):
        z_v.at[i][pl.ds(j * L, L)] = zero
      return carry
    lax.fori_loop(0, SROWS, zrow, 0)
    pltpu.sync_copy(z_v, acc_sh.at[pl.ds(sid * SROWS, SROWS)])
    plsc.subcore_barrier()

    for j in range(K):
      pltpu.make_async_copy(
          b_hbm.at[pl.ds(0, CL)], ids_v.at[j], sid_sem).wait()

    def mask_row(j):
      c = wid + NW * j
      st = chunk_start(j)
      for g in range(CL // L):
        gl = st + g * L + lanes
        v = ids_v.at[j][pl.ds(g * L, L)]
        ids_v.at[j][pl.ds(g * L, L)] = jnp.where(gl >= c * CL, v, S)

    def issue_stage(j):
      b = j % NBUF
      pltpu.async_copy(
          x_hbm.at[pl.ds(chunk_start(j), CL)], bufs[b], ssems[b])

    def wait_stage(j):
      b = j % NBUF
      pltpu.make_async_copy(
          x_hbm.at[pl.ds(0, CL)], bufs[b], ssems[b]).wait()

    def issue_scatter(j):
      b = j % NBUF
      pltpu.async_copy(bufs[b], acc_sh.at[ids_v.at[j]], csems[b], add=True)

    def wait_scatter(j):
      b = j % NBUF
      pltpu.make_async_copy(
          bufs[b], acc_sh.at[ids_v.at[j]], csems[b]).wait()

    LOOK = NBUF // 2
    for j in range(min(NBUF, K)):
      issue_stage(j)
    for j in range(K):
      wait_stage(j)
      mask_row(j)
      issue_scatter(j)
      if j - LOOK >= 0 and j + LOOK < K:
        wait_scatter(j - LOOK)
        issue_stage(j + LOOK)
    for j in range(max(K - 2 * LOOK, 0), K):
      wait_scatter(j)

    plsc.subcore_barrier()
    pltpu.sync_copy(acc_sh.at[pl.ds(sid * SROWS, SROWS)], z_v)
    pltpu.sync_copy(z_v, out_hbm.at[cid, pl.ds(sid * SROWS, SROWS)])

  return k(x, batch)


def _combine(parts):
  def body(p_ref, o_ref):
    o_ref[...] = p_ref[0] + p_ref[1]

  return pl.pallas_call(
      body,
      out_shape=jax.ShapeDtypeStruct((S, D), jnp.float32),
  )(parts)


@jax.jit
def kernel(x, batch):
  parts = _sc_partial_sums(x, batch.astype(jnp.int32))
  return _combine(parts)
